# baseline (speedup 1.0000x reference)
import functools

import jax
import jax.numpy as jnp
from jax import lax
from jax.experimental import pallas as pl
from jax.experimental.pallas import tpu as pltpu
from jax.experimental.pallas import tpu_sc as plsc

_info = plsc.get_sparse_core_info()
_NC, _NS = _info.num_cores, _info.num_subcores
_NW = _NC * _NS

_DP = 128
_CH = 128
_NRING = 5
_LEAD = 3


def _sc_gather(table_padded, idx_flat):
    n = idx_flat.shape[0]
    toks_per_w = n // _NW
    nchunks = toks_per_w // _CH
    mesh = plsc.VectorSubcoreMesh(core_axis_name="c", subcore_axis_name="s")

    @functools.partial(
        pl.kernel,
        mesh=mesh,
        out_type=jax.ShapeDtypeStruct((n, _DP), jnp.float32),
        scratch_types=[
            pltpu.VMEM((toks_per_w,), jnp.int32),
            pltpu.VMEM((_NRING, _CH, _DP), jnp.float32),
            pltpu.SemaphoreType.DMA((_NRING,)),
            pltpu.SemaphoreType.DMA((_NRING,)),
        ],
    )
    def k(table_hbm, idx_hbm, out_hbm, idx_v, rows_v, gsem, ssem):
        wid = lax.axis_index("s") * _NC + lax.axis_index("c")
        base = wid * toks_per_w
        pltpu.sync_copy(idx_hbm.at[pl.ds(base, toks_per_w)], idx_v)

        def gather(c, p):
            return pltpu.make_async_copy(
                table_hbm.at[idx_v.at[pl.ds(c * _CH, _CH)]],
                rows_v.at[p], gsem.at[p])

        def store(c, p):
            return pltpu.make_async_copy(
                rows_v.at[p], out_hbm.at[pl.ds(base + c * _CH, _CH)],
                ssem.at[p])

        for p in range(_LEAD):
            gather(p, p).start()

        def body(c, p):
            gather(c, p).wait()
            store(c, p).start()
            np_ = (p + _LEAD) % _NRING

            @pl.when(c >= _NRING - _LEAD)
            def _():
                store(c - (_NRING - _LEAD), np_).wait()

            @pl.when(c + _LEAD < nchunks)
            def _():
                gather(c + _LEAD, np_).start()

        def outer(gg, carry):
            for j in range(_NRING):
                body(gg * _NRING + j, j)
            return carry

        lax.fori_loop(0, nchunks // _NRING, outer, 0)

        for c in range((nchunks // _NRING) * _NRING, nchunks):
            body(c, c % _NRING)

        for c in range(nchunks - (_NRING - _LEAD), nchunks):
            store(c, c % _NRING).wait()

    return k(table_padded, idx_flat)


def kernel(token_ids, embedding_matrix):
    b, t = token_ids.shape
    d = embedding_matrix.shape[1]
    table_padded = jnp.concatenate([embedding_matrix, embedding_matrix], axis=1)
    idx_flat = token_ids.astype(jnp.int32).reshape(-1)
    out_padded = _sc_gather(table_padded, idx_flat)
    return out_padded.reshape(b, t, _DP)[:, :, :d]
